# baseline (speedup 1.0000x reference)
import functools

import jax
import jax.numpy as jnp
from jax import lax
from jax.experimental import pallas as pl
from jax.experimental.pallas import tpu as pltpu
from jax.experimental.pallas import tpu_sc as plsc

N = 50000
NP = 51200
G = 64
E = 800000
EP = 851968
EPS = 1e-5

NC = 2
NS = 16
NW = NC * NS
EPT = EP // NW
K = 16
NOC = EPT // (K * 128)
RW = 256
RPT = NP // NS

RB = 2048
NB = NP // RB

_mesh = plsc.VectorSubcoreMesh(core_axis_name="c", subcore_axis_name="s")
_SC_PARAMS = pltpu.CompilerParams(use_tc_tiling_on_sc=False)


def _deg_body(dst2d, zeros_hbm, ones_hbm, deg_out, shared, didx, ones_v):
    c = lax.axis_index("c")
    s = lax.axis_index("s")
    w = c * NS + s
    pltpu.sync_copy(ones_hbm.at[pl.ds(0, 128)], ones_v)
    pltpu.sync_copy(zeros_hbm.at[pl.ds(0, RPT)],
                    shared.at[pl.ds(s * RPT, RPT)])
    plsc.subcore_barrier()

    def chunk(i, carry):
        row0 = w * (NOC * K) + i * K
        pltpu.sync_copy(dst2d.at[pl.ds(row0, K)], didx)
        for j in range(K):
            pltpu.sync_copy(ones_v, shared.at[didx.at[j]], add=True)
        return carry

    lax.fori_loop(0, NOC, chunk, 0)
    plsc.subcore_barrier()
    pltpu.sync_copy(shared.at[pl.ds(s * RPT, RPT)],
                    deg_out.at[pl.ds(s * RPT, RPT), pl.ds(8 * c, 8)])


_deg_kernel = functools.partial(
    pl.kernel,
    out_type=jax.ShapeDtypeStruct((NP, 128), jnp.float32),
    mesh=_mesh,
    compiler_params=_SC_PARAMS,
    scratch_types=[
        pltpu.VMEM_SHARED((NP, 8), jnp.float32),
        pltpu.VMEM((K, 128), jnp.int32),
        pltpu.VMEM((128, 8), jnp.float32),
    ],
)(_deg_body)


def _make_prop(F, nsweep):
    KK = 104 if F == 16 else 16
    NCH = 208 // KK
    NWAVES = KK // 2

    def body(*args):
        table = args[0]
        srcs = args[1:1 + nsweep]
        dst2d, zeros_hbm, acc_out = args[1 + nsweep:4 + nsweep]
        (shared, sidx, didx, rows0, rows1,
         gsem0, gsem1, ssem0, ssem1) = args[4 + nsweep:]
        c = lax.axis_index("c")
        s = lax.axis_index("s")
        w = c * NS + s
        bufs = (rows0, rows1)
        gsems = (gsem0, gsem1)
        ssems = (ssem0, ssem1)

        def sweep(t, src2d):
            pltpu.sync_copy(zeros_hbm.at[pl.ds(0, RPT)],
                            shared.at[pl.ds(s * RPT, RPT)])
            plsc.subcore_barrier()

            def chunk(i, carry):
                row0 = w * 208 + i * KK
                pltpu.sync_copy(src2d.at[pl.ds(row0, KK)], sidx)
                pltpu.sync_copy(dst2d.at[pl.ds(row0, KK)], didx)
                gd = {}

                def issue_g(wv):
                    p = wv % 2
                    gd[wv] = [pltpu.async_copy(
                        table.at[sidx.at[wv * 2 + j]],
                        bufs[p].at[pl.ds(j * 128, 128)], gsems[p])
                        for j in range(2)]

                def issue_s(wv):
                    p = wv % 2
                    for j in range(2):
                        pltpu.async_copy(bufs[p].at[pl.ds(j * 128, 128)],
                                         shared.at[didx.at[wv * 2 + j]],
                                         ssems[p], add=True)

                def drain_s(wv):
                    p = wv % 2
                    for j in range(2):
                        pltpu.make_async_copy(
                            bufs[p].at[pl.ds(j * 128, 128)],
                            shared.at[didx.at[wv * 2 + j]], ssems[p]).wait()

                issue_g(0)
                for wv in range(NWAVES):
                    if wv + 1 < NWAVES:
                        if wv >= 1:
                            drain_s(wv - 1)
                        issue_g(wv + 1)
                    for g in gd[wv]:
                        g.wait()
                    issue_s(wv)
                drain_s(NWAVES - 2)
                drain_s(NWAVES - 1)
                return carry

            lax.fori_loop(0, NCH, chunk, 0)
            plsc.subcore_barrier()
            lane0 = (c * nsweep + t) * F
            pltpu.sync_copy(
                shared.at[pl.ds(s * RPT, RPT)],
                acc_out.at[pl.ds(s * RPT, RPT), pl.ds(lane0, F)])
            plsc.subcore_barrier()

        for t in range(nsweep):
            sweep(t, srcs[t])

    return pl.kernel(
        body,
        out_type=jax.ShapeDtypeStruct((NP, 128), jnp.float32),
        mesh=_mesh,
        compiler_params=_SC_PARAMS,
        scratch_types=[
            pltpu.VMEM_SHARED((NP, F), jnp.float32),
            pltpu.VMEM((KK, 128), jnp.int32),
            pltpu.VMEM((KK, 128), jnp.int32),
            pltpu.VMEM((RW, F), jnp.float32),
            pltpu.VMEM((RW, F), jnp.float32),
            pltpu.SemaphoreType.DMA,
            pltpu.SemaphoreType.DMA,
            pltpu.SemaphoreType.DMA,
            pltpu.SemaphoreType.DMA,
        ],
    )


_prop16 = _make_prop(16, 1)
_prop32 = _make_prop(32, 2)


def _row_spec(fdim):
    return pl.BlockSpec((RB, fdim), lambda j: (j, 0))


_FULL = lambda shape: pl.BlockSpec(shape, lambda j: tuple(0 for _ in shape))


def _xs_body(deg_ref, x_ref, o_ref):
    dblk = deg_ref[...]
    dinv = lax.rsqrt(dblk[:, 0:1] + dblk[:, 8:9] + 1.0)
    lanes = jax.lax.broadcasted_iota(jnp.int32, (1, 128), 1)
    px = jnp.concatenate(
        [x_ref[...] * dinv, jnp.zeros((RB, 118), jnp.float32)], axis=1)
    o_ref[...] = jnp.where(lanes == 10, dinv, px)


def _tc_xs(deg, x_p):
    return pl.pallas_call(
        _xs_body,
        grid=(NB,),
        in_specs=[_row_spec(128), _row_spec(10)],
        out_specs=_row_spec(128),
        out_shape=jax.ShapeDtypeStruct((NP, 128), jnp.float32),
    )(deg, x_p)


def _l1_body(acc_ref, xs_ref, w_ref, b_ref, z_ref, s_ref, q_ref):
    j = pl.program_id(0)
    xs = xs_ref[...]
    dinv = xs[:, 10:11]
    p = (acc_ref[:, 0:16] + acc_ref[:, 16:32] + xs[:, 0:16]) * dinv
    z = jnp.dot(p, w_ref[...], preferred_element_type=jnp.float32) + b_ref[...]
    z_ref[...] = jnp.concatenate(
        [z, dinv, jnp.zeros((RB, 63), jnp.float32)], axis=1)
    rows = jax.lax.broadcasted_iota(jnp.int32, (RB, 1), 0) + j * RB
    zm = jnp.where(rows < N, z, 0.0)
    ps = jnp.sum(zm, axis=0, keepdims=True)
    pq = jnp.sum(zm * zm, axis=0, keepdims=True)

    @pl.when(j == 0)
    def _():
        s_ref[...] = ps
        q_ref[...] = pq

    @pl.when(j > 0)
    def _():
        s_ref[...] += ps
        q_ref[...] += pq


def _tc_l1(acc1, xs, w1p, b1r):
    return pl.pallas_call(
        _l1_body,
        grid=(NB,),
        in_specs=[_row_spec(128), _row_spec(128),
                  _FULL((16, 64)), _FULL((1, 64))],
        out_specs=[_row_spec(128),
                   _FULL((1, 64)), _FULL((1, 64))],
        out_shape=[jax.ShapeDtypeStruct((NP, 128), jnp.float32),
                   jax.ShapeDtypeStruct((1, 64), jnp.float32),
                   jax.ShapeDtypeStruct((1, 64), jnp.float32)],
    )(acc1, xs, w1p, b1r)


def _h1_body(z_ref, s_ref, q_ref, g_ref, be_ref, o_ref):
    mean = s_ref[...] / N
    var = q_ref[...] / N - mean * mean
    scale = g_ref[...] * lax.rsqrt(var + EPS)
    shift = be_ref[...] - mean * scale
    zblk = z_ref[...]
    dinv = zblk[:, 64:65]
    h = jnp.maximum(zblk[:, 0:64] * scale + shift, 0.0) * dinv
    zpad = jnp.zeros((RB, 31), jnp.float32)
    o_ref[...] = jnp.concatenate(
        [h[:, 0:32], dinv, zpad, h[:, 32:64],
         jnp.zeros((RB, 32), jnp.float32)], axis=1)


def _tc_h1(z1, s1, q1, g1r, be1r):
    return pl.pallas_call(
        _h1_body,
        grid=(NB,),
        in_specs=[_row_spec(128),
                  _FULL((1, 64)), _FULL((1, 64)),
                  _FULL((1, 64)), _FULL((1, 64))],
        out_specs=_row_spec(128),
        out_shape=jax.ShapeDtypeStruct((NP, 128), jnp.float32),
    )(z1, s1, q1, g1r, be1r)


def _l2_body(acc_ref, h_ref, w_ref, b_ref, z_ref, s_ref, q_ref):
    j = pl.program_id(0)
    acc = acc_ref[...]
    h = h_ref[...]
    dinv = h[:, 32:33]
    pa = (acc[:, 0:32] + acc[:, 64:96] + h[:, 0:32]) * dinv
    pb = (acc[:, 32:64] + acc[:, 96:128] + h[:, 64:96]) * dinv
    w = w_ref[...]
    z = (jnp.dot(pa, w[:32, :], preferred_element_type=jnp.float32)
         + jnp.dot(pb, w[32:, :], preferred_element_type=jnp.float32)
         + b_ref[...])
    z_ref[...] = z
    rows = jax.lax.broadcasted_iota(jnp.int32, (RB, 1), 0) + j * RB
    zm = jnp.where(rows < N, z, 0.0)
    ps = jnp.sum(zm, axis=0, keepdims=True)
    pq = jnp.sum(zm * zm, axis=0, keepdims=True)

    @pl.when(j == 0)
    def _():
        s_ref[...] = ps
        q_ref[...] = pq

    @pl.when(j > 0)
    def _():
        s_ref[...] += ps
        q_ref[...] += pq


def _tc_l2(acc2, h1, w2, b2r):
    return pl.pallas_call(
        _l2_body,
        grid=(NB,),
        in_specs=[_row_spec(128), _row_spec(128),
                  _FULL((64, 128)), _FULL((1, 128))],
        out_specs=[_row_spec(128), _FULL((1, 128)), _FULL((1, 128))],
        out_shape=[jax.ShapeDtypeStruct((NP, 128), jnp.float32),
                   jax.ShapeDtypeStruct((1, 128), jnp.float32),
                   jax.ShapeDtypeStruct((1, 128), jnp.float32)],
    )(acc2, h1, w2, b2r)


def _pool_body(z_ref, s_ref, q_ref, g_ref, be_ref, o0_ref, o1_ref,
               fw1_ref, fb1_ref, fw2_ref, fb2_ref, o_ref, pacc, cacc):
    j = pl.program_id(0)

    @pl.when(j == 0)
    def _():
        pacc[...] = jnp.zeros_like(pacc)
        cacc[...] = jnp.zeros_like(cacc)

    mean = s_ref[...] / N
    var = q_ref[...] / N - mean * mean
    scale = g_ref[...] * lax.rsqrt(var + EPS)
    shift = be_ref[...] - mean * scale
    h = jnp.maximum(z_ref[...] * scale + shift, 0.0)
    rows = jax.lax.broadcasted_iota(jnp.int32, (RB, 1), 0) + j * RB
    oh = ((rows >= o0_ref[...]) & (rows < o1_ref[...])).astype(jnp.float32)
    dn = (((0,), (0,)), ((), ()))
    pacc[...] += jax.lax.dot_general(oh, h, dn,
                                     preferred_element_type=jnp.float32)
    cacc[...] += jax.lax.dot_general(oh, jnp.ones((RB, 1), jnp.float32), dn,
                                     preferred_element_type=jnp.float32)

    @pl.when(j == NB - 1)
    def _():
        pool = pacc[...] / jnp.maximum(cacc[...], 1.0)
        t = jnp.maximum(
            jnp.dot(pool, fw1_ref[...], preferred_element_type=jnp.float32)
            + fb1_ref[...], 0.0)
        o_ref[...] = (jnp.dot(t, fw2_ref[...],
                              preferred_element_type=jnp.float32)
                      + fb2_ref[...])


def _tc_pool(z2, s2, q2, g2r, be2r, off0, off1, fw1, fb1r, fw2, fb2r):
    return pl.pallas_call(
        _pool_body,
        grid=(NB,),
        in_specs=[_row_spec(128), _FULL((1, 128)), _FULL((1, 128)),
                  _FULL((1, 128)), _FULL((1, 128)),
                  _FULL((1, G)), _FULL((1, G)),
                  _FULL((128, 64)), _FULL((1, 64)), _FULL((64, 1)),
                  _FULL((1, 1))],
        out_specs=_FULL((G, 1)),
        out_shape=jax.ShapeDtypeStruct((G, 1), jnp.float32),
        scratch_shapes=[pltpu.VMEM((G, 128), jnp.float32),
                        pltpu.VMEM((G, 1), jnp.float32)],
    )(z2, s2, q2, g2r, be2r, off0, off1, fw1, fb1r, fw2, fb2r)


def kernel(x, edge_index, batch, W1, b1, g1, be1, W2, b2, g2, be2,
           fw1, fb1, fw2, fb2):
    pad_ids = (N + (jnp.arange(EP - E, dtype=jnp.int32) % (NP - N)))
    src = jnp.concatenate([edge_index[0], pad_ids])
    dst2d = jnp.concatenate([edge_index[1], pad_ids]).reshape(EP // 128, 128)
    src8 = (src * 8).reshape(EP // 128, 128)
    src4a = (src * 4).reshape(EP // 128, 128)
    src4b = (src * 4 + 2).reshape(EP // 128, 128)
    x_p = jnp.pad(x, ((0, NP - N), (0, 0)))
    z8 = jnp.zeros((RPT, 8), jnp.float32)
    zp16 = jnp.zeros((RPT, 16), jnp.float32)
    zp32 = jnp.zeros((RPT, 32), jnp.float32)
    ones8 = jnp.ones((128, 128), jnp.float32).reshape(2048, 8)
    off = jnp.searchsorted(batch, jnp.arange(G + 1, dtype=jnp.int32)
                           ).astype(jnp.int32)
    off0, off1 = off[:G].reshape(1, G), off[1:].reshape(1, G)
    w1p = jnp.pad(W1, ((0, 6), (0, 0)))
    b1r, g1r, be1r = b1.reshape(1, 64), g1.reshape(1, 64), be1.reshape(1, 64)
    b2r, g2r, be2r = b2.reshape(1, 128), g2.reshape(1, 128), be2.reshape(1, 128)
    fb1r, fb2r = fb1.reshape(1, 64), fb2.reshape(1, 1)

    deg = _deg_kernel(dst2d, z8, ones8)
    xs = _tc_xs(deg, x_p)

    acc1 = _prop16(xs.reshape(8 * NP, 16), src8, dst2d, zp16)
    z1, s1, q1 = _tc_l1(acc1, xs, w1p, b1r)
    h1 = _tc_h1(z1, s1, q1, g1r, be1r)

    acc2 = _prop32(h1.reshape(4 * NP, 32), src4a, src4b, dst2d, zp32)
    z2, s2, q2 = _tc_l2(acc2, h1, W2, b2r)

    return _tc_pool(z2, s2, q2, g2r, be2r, off0, off1, fw1, fb1r, fw2, fb2r)


# speedup vs baseline: 1.0259x; 1.0259x over previous
import functools

import jax
import jax.numpy as jnp
from jax import lax
from jax.experimental import pallas as pl
from jax.experimental.pallas import tpu as pltpu
from jax.experimental.pallas import tpu_sc as plsc

N = 50000
NP = 51200
G = 64
E = 800000
EP = 851968
EPS = 1e-5

NC = 2
NS = 16
NW = NC * NS
EPT = EP // NW
K = 16
NOC = EPT // (K * 128)
RW = 256
RPT = NP // NS

RB = 2048
NB = NP // RB

_mesh = plsc.VectorSubcoreMesh(core_axis_name="c", subcore_axis_name="s")
_SC_PARAMS = pltpu.CompilerParams(use_tc_tiling_on_sc=False)


def _deg_body(dst2d, zeros_hbm, ones_hbm, deg_out, shared, didx, ones_v,
              ssem):
    c = lax.axis_index("c")
    s = lax.axis_index("s")
    w = c * NS + s
    pltpu.sync_copy(ones_hbm.at[pl.ds(0, 128)], ones_v)
    pltpu.sync_copy(zeros_hbm.at[pl.ds(0, RPT)],
                    shared.at[pl.ds(s * RPT, RPT)])
    plsc.subcore_barrier()

    def chunk(i, carry):
        row0 = w * 208 + i * 104
        pltpu.sync_copy(dst2d.at[pl.ds(row0, 104)], didx)

        def issue_grp(g):
            for j in range(8):
                pltpu.async_copy(ones_v, shared.at[didx.at[g * 8 + j]],
                                 ssem, add=True)

        def drain_grp(g):
            for j in range(8):
                pltpu.make_async_copy(ones_v, shared.at[didx.at[g * 8 + j]],
                                      ssem).wait()

        for g in range(13):
            if g >= 2:
                drain_grp(g - 2)
            issue_grp(g)
        drain_grp(11)
        drain_grp(12)
        return carry

    lax.fori_loop(0, 2, chunk, 0)
    plsc.subcore_barrier()
    pltpu.sync_copy(shared.at[pl.ds(s * RPT, RPT)],
                    deg_out.at[pl.ds(s * RPT, RPT), pl.ds(8 * c, 8)])


_deg_kernel = functools.partial(
    pl.kernel,
    out_type=jax.ShapeDtypeStruct((NP, 128), jnp.float32),
    mesh=_mesh,
    compiler_params=_SC_PARAMS,
    scratch_types=[
        pltpu.VMEM_SHARED((NP, 8), jnp.float32),
        pltpu.VMEM((104, 128), jnp.int32),
        pltpu.VMEM((128, 8), jnp.float32),
        pltpu.SemaphoreType.DMA,
    ],
)(_deg_body)


def _make_prop(F, nsweep):
    KK = 104 if F == 16 else 16
    NCH = 208 // KK
    NWAVES = KK // 2

    def body(*args):
        table = args[0]
        srcs = args[1:1 + nsweep]
        dst2d, zeros_hbm, acc_out = args[1 + nsweep:4 + nsweep]
        (shared, sidx, didx, rows0, rows1,
         gsem0, gsem1, ssem0, ssem1) = args[4 + nsweep:]
        c = lax.axis_index("c")
        s = lax.axis_index("s")
        w = c * NS + s
        bufs = (rows0, rows1)
        gsems = (gsem0, gsem1)
        ssems = (ssem0, ssem1)

        def sweep(t, src2d):
            pltpu.sync_copy(zeros_hbm.at[pl.ds(0, RPT)],
                            shared.at[pl.ds(s * RPT, RPT)])
            plsc.subcore_barrier()

            def chunk(i, carry):
                row0 = w * 208 + i * KK
                pltpu.sync_copy(src2d.at[pl.ds(row0, KK)], sidx)
                pltpu.sync_copy(dst2d.at[pl.ds(row0, KK)], didx)
                gd = {}

                def issue_g(wv):
                    p = wv % 2
                    gd[wv] = [pltpu.async_copy(
                        table.at[sidx.at[wv * 2 + j]],
                        bufs[p].at[pl.ds(j * 128, 128)], gsems[p])
                        for j in range(2)]

                def issue_s(wv):
                    p = wv % 2
                    for j in range(2):
                        pltpu.async_copy(bufs[p].at[pl.ds(j * 128, 128)],
                                         shared.at[didx.at[wv * 2 + j]],
                                         ssems[p], add=True)

                def drain_s(wv):
                    p = wv % 2
                    for j in range(2):
                        pltpu.make_async_copy(
                            bufs[p].at[pl.ds(j * 128, 128)],
                            shared.at[didx.at[wv * 2 + j]], ssems[p]).wait()

                issue_g(0)
                for wv in range(NWAVES):
                    if wv + 1 < NWAVES:
                        if wv >= 1:
                            drain_s(wv - 1)
                        issue_g(wv + 1)
                    for g in gd[wv]:
                        g.wait()
                    issue_s(wv)
                drain_s(NWAVES - 2)
                drain_s(NWAVES - 1)
                return carry

            lax.fori_loop(0, NCH, chunk, 0)
            plsc.subcore_barrier()
            lane0 = (c * nsweep + t) * F
            pltpu.sync_copy(
                shared.at[pl.ds(s * RPT, RPT)],
                acc_out.at[pl.ds(s * RPT, RPT), pl.ds(lane0, F)])
            plsc.subcore_barrier()

        for t in range(nsweep):
            sweep(t, srcs[t])

    return pl.kernel(
        body,
        out_type=jax.ShapeDtypeStruct((NP, 128), jnp.float32),
        mesh=_mesh,
        compiler_params=_SC_PARAMS,
        scratch_types=[
            pltpu.VMEM_SHARED((NP, F), jnp.float32),
            pltpu.VMEM((KK, 128), jnp.int32),
            pltpu.VMEM((KK, 128), jnp.int32),
            pltpu.VMEM((RW, F), jnp.float32),
            pltpu.VMEM((RW, F), jnp.float32),
            pltpu.SemaphoreType.DMA,
            pltpu.SemaphoreType.DMA,
            pltpu.SemaphoreType.DMA,
            pltpu.SemaphoreType.DMA,
        ],
    )


_prop16 = _make_prop(16, 1)
_prop32 = _make_prop(32, 2)


def _row_spec(fdim):
    return pl.BlockSpec((RB, fdim), lambda j: (j, 0))


_FULL = lambda shape: pl.BlockSpec(shape, lambda j: tuple(0 for _ in shape))


def _xs_body(deg_ref, x_ref, o_ref):
    dblk = deg_ref[...]
    dinv = lax.rsqrt(dblk[:, 0:1] + dblk[:, 8:9] + 1.0)
    lanes = jax.lax.broadcasted_iota(jnp.int32, (1, 128), 1)
    px = jnp.concatenate(
        [x_ref[...] * dinv, jnp.zeros((RB, 118), jnp.float32)], axis=1)
    o_ref[...] = jnp.where(lanes == 10, dinv, px)


def _tc_xs(deg, x_p):
    return pl.pallas_call(
        _xs_body,
        grid=(NB,),
        in_specs=[_row_spec(128), _row_spec(10)],
        out_specs=_row_spec(128),
        out_shape=jax.ShapeDtypeStruct((NP, 128), jnp.float32),
    )(deg, x_p)


def _l1_body(acc_ref, xs_ref, w_ref, b_ref, z_ref, s_ref, q_ref):
    j = pl.program_id(0)
    xs = xs_ref[...]
    dinv = xs[:, 10:11]
    p = (acc_ref[:, 0:16] + acc_ref[:, 16:32] + xs[:, 0:16]) * dinv
    z = jnp.dot(p, w_ref[...], preferred_element_type=jnp.float32) + b_ref[...]
    z_ref[...] = jnp.concatenate(
        [z, dinv, jnp.zeros((RB, 63), jnp.float32)], axis=1)
    rows = jax.lax.broadcasted_iota(jnp.int32, (RB, 1), 0) + j * RB
    zm = jnp.where(rows < N, z, 0.0)
    ps = jnp.sum(zm, axis=0, keepdims=True)
    pq = jnp.sum(zm * zm, axis=0, keepdims=True)

    @pl.when(j == 0)
    def _():
        s_ref[...] = ps
        q_ref[...] = pq

    @pl.when(j > 0)
    def _():
        s_ref[...] += ps
        q_ref[...] += pq


def _tc_l1(acc1, xs, w1p, b1r):
    return pl.pallas_call(
        _l1_body,
        grid=(NB,),
        in_specs=[_row_spec(128), _row_spec(128),
                  _FULL((16, 64)), _FULL((1, 64))],
        out_specs=[_row_spec(128),
                   _FULL((1, 64)), _FULL((1, 64))],
        out_shape=[jax.ShapeDtypeStruct((NP, 128), jnp.float32),
                   jax.ShapeDtypeStruct((1, 64), jnp.float32),
                   jax.ShapeDtypeStruct((1, 64), jnp.float32)],
    )(acc1, xs, w1p, b1r)


def _h1_body(z_ref, s_ref, q_ref, g_ref, be_ref, o_ref):
    mean = s_ref[...] / N
    var = q_ref[...] / N - mean * mean
    scale = g_ref[...] * lax.rsqrt(var + EPS)
    shift = be_ref[...] - mean * scale
    zblk = z_ref[...]
    dinv = zblk[:, 64:65]
    h = jnp.maximum(zblk[:, 0:64] * scale + shift, 0.0) * dinv
    zpad = jnp.zeros((RB, 31), jnp.float32)
    o_ref[...] = jnp.concatenate(
        [h[:, 0:32], dinv, zpad, h[:, 32:64],
         jnp.zeros((RB, 32), jnp.float32)], axis=1)


def _tc_h1(z1, s1, q1, g1r, be1r):
    return pl.pallas_call(
        _h1_body,
        grid=(NB,),
        in_specs=[_row_spec(128),
                  _FULL((1, 64)), _FULL((1, 64)),
                  _FULL((1, 64)), _FULL((1, 64))],
        out_specs=_row_spec(128),
        out_shape=jax.ShapeDtypeStruct((NP, 128), jnp.float32),
    )(z1, s1, q1, g1r, be1r)


def _l2_body(acc_ref, h_ref, w_ref, b_ref, z_ref, s_ref, q_ref):
    j = pl.program_id(0)
    acc = acc_ref[...]
    h = h_ref[...]
    dinv = h[:, 32:33]
    pa = (acc[:, 0:32] + acc[:, 64:96] + h[:, 0:32]) * dinv
    pb = (acc[:, 32:64] + acc[:, 96:128] + h[:, 64:96]) * dinv
    w = w_ref[...]
    z = (jnp.dot(pa, w[:32, :], preferred_element_type=jnp.float32)
         + jnp.dot(pb, w[32:, :], preferred_element_type=jnp.float32)
         + b_ref[...])
    z_ref[...] = z
    rows = jax.lax.broadcasted_iota(jnp.int32, (RB, 1), 0) + j * RB
    zm = jnp.where(rows < N, z, 0.0)
    ps = jnp.sum(zm, axis=0, keepdims=True)
    pq = jnp.sum(zm * zm, axis=0, keepdims=True)

    @pl.when(j == 0)
    def _():
        s_ref[...] = ps
        q_ref[...] = pq

    @pl.when(j > 0)
    def _():
        s_ref[...] += ps
        q_ref[...] += pq


def _tc_l2(acc2, h1, w2, b2r):
    return pl.pallas_call(
        _l2_body,
        grid=(NB,),
        in_specs=[_row_spec(128), _row_spec(128),
                  _FULL((64, 128)), _FULL((1, 128))],
        out_specs=[_row_spec(128), _FULL((1, 128)), _FULL((1, 128))],
        out_shape=[jax.ShapeDtypeStruct((NP, 128), jnp.float32),
                   jax.ShapeDtypeStruct((1, 128), jnp.float32),
                   jax.ShapeDtypeStruct((1, 128), jnp.float32)],
    )(acc2, h1, w2, b2r)


def _pool_body(z_ref, s_ref, q_ref, g_ref, be_ref, o0_ref, o1_ref,
               fw1_ref, fb1_ref, fw2_ref, fb2_ref, o_ref, pacc, cacc):
    j = pl.program_id(0)

    @pl.when(j == 0)
    def _():
        pacc[...] = jnp.zeros_like(pacc)
        cacc[...] = jnp.zeros_like(cacc)

    mean = s_ref[...] / N
    var = q_ref[...] / N - mean * mean
    scale = g_ref[...] * lax.rsqrt(var + EPS)
    shift = be_ref[...] - mean * scale
    h = jnp.maximum(z_ref[...] * scale + shift, 0.0)
    rows = jax.lax.broadcasted_iota(jnp.int32, (RB, 1), 0) + j * RB
    oh = ((rows >= o0_ref[...]) & (rows < o1_ref[...])).astype(jnp.float32)
    dn = (((0,), (0,)), ((), ()))
    pacc[...] += jax.lax.dot_general(oh, h, dn,
                                     preferred_element_type=jnp.float32)
    cacc[...] += jax.lax.dot_general(oh, jnp.ones((RB, 1), jnp.float32), dn,
                                     preferred_element_type=jnp.float32)

    @pl.when(j == NB - 1)
    def _():
        pool = pacc[...] / jnp.maximum(cacc[...], 1.0)
        t = jnp.maximum(
            jnp.dot(pool, fw1_ref[...], preferred_element_type=jnp.float32)
            + fb1_ref[...], 0.0)
        o_ref[...] = (jnp.dot(t, fw2_ref[...],
                              preferred_element_type=jnp.float32)
                      + fb2_ref[...])


def _tc_pool(z2, s2, q2, g2r, be2r, off0, off1, fw1, fb1r, fw2, fb2r):
    return pl.pallas_call(
        _pool_body,
        grid=(NB,),
        in_specs=[_row_spec(128), _FULL((1, 128)), _FULL((1, 128)),
                  _FULL((1, 128)), _FULL((1, 128)),
                  _FULL((1, G)), _FULL((1, G)),
                  _FULL((128, 64)), _FULL((1, 64)), _FULL((64, 1)),
                  _FULL((1, 1))],
        out_specs=_FULL((G, 1)),
        out_shape=jax.ShapeDtypeStruct((G, 1), jnp.float32),
        scratch_shapes=[pltpu.VMEM((G, 128), jnp.float32),
                        pltpu.VMEM((G, 1), jnp.float32)],
    )(z2, s2, q2, g2r, be2r, off0, off1, fw1, fb1r, fw2, fb2r)


def kernel(x, edge_index, batch, W1, b1, g1, be1, W2, b2, g2, be2,
           fw1, fb1, fw2, fb2):
    pad_ids = (N + (jnp.arange(EP - E, dtype=jnp.int32) % (NP - N)))
    src = jnp.concatenate([edge_index[0], pad_ids])
    dst2d = jnp.concatenate([edge_index[1], pad_ids]).reshape(EP // 128, 128)
    src8 = (src * 8).reshape(EP // 128, 128)
    src4a = (src * 4).reshape(EP // 128, 128)
    src4b = (src * 4 + 2).reshape(EP // 128, 128)
    x_p = jnp.pad(x, ((0, NP - N), (0, 0)))
    z8 = jnp.zeros((RPT, 8), jnp.float32)
    zp16 = jnp.zeros((RPT, 16), jnp.float32)
    zp32 = jnp.zeros((RPT, 32), jnp.float32)
    ones8 = jnp.ones((128, 128), jnp.float32).reshape(2048, 8)
    off = jnp.searchsorted(batch, jnp.arange(G + 1, dtype=jnp.int32)
                           ).astype(jnp.int32)
    off0, off1 = off[:G].reshape(1, G), off[1:].reshape(1, G)
    w1p = jnp.pad(W1, ((0, 6), (0, 0)))
    b1r, g1r, be1r = b1.reshape(1, 64), g1.reshape(1, 64), be1.reshape(1, 64)
    b2r, g2r, be2r = b2.reshape(1, 128), g2.reshape(1, 128), be2.reshape(1, 128)
    fb1r, fb2r = fb1.reshape(1, 64), fb2.reshape(1, 1)

    deg = _deg_kernel(dst2d, z8, ones8)
    xs = _tc_xs(deg, x_p)

    acc1 = _prop16(xs.reshape(8 * NP, 16), src8, dst2d, zp16)
    z1, s1, q1 = _tc_l1(acc1, xs, w1p, b1r)
    h1 = _tc_h1(z1, s1, q1, g1r, be1r)

    acc2 = _prop32(h1.reshape(4 * NP, 32), src4a, src4b, dst2d, zp32)
    z2, s2, q2 = _tc_l2(acc2, h1, W2, b2r)

    return _tc_pool(z2, s2, q2, g2r, be2r, off0, off1, fw1, fb1r, fw2, fb2r)
